# Initial kernel scaffold; baseline (speedup 1.0000x reference)
import math

import jax
import jax.numpy as jnp
from jax import lax
from jax.experimental import pallas as pl

_BASE = 2
_BT = 2048


def _body(emb_ref, out_ref):
    l = pl.program_id(0)
    e = emb_ref[...]
    bt, d = out_ref.shape[1], out_ref.shape[2]
    tpar = lax.broadcasted_iota(jnp.int32, (bt, d), 0) % 2
    use_second = jnp.logical_and(tpar == 1, (l % _BASE) == 0)
    out_ref[0, :, :] = jnp.where(use_second, e[1:2, :], e[0:1, :])


def kernel(x, emb):
    B, T = x.shape
    del B
    max_level = int(math.ceil(math.log(T, _BASE)))
    d = emb.shape[1]
    bt = min(_BT, T)
    grid = (max_level, T // bt)
    return pl.pallas_call(
        _body,
        grid=grid,
        in_specs=[pl.BlockSpec((_BASE, d), lambda l, j: (l, 0))],
        out_specs=pl.BlockSpec((1, bt, d), lambda l, j: (l, j, 0)),
        out_shape=jax.ShapeDtypeStruct((max_level, T, d), emb.dtype),
    )(emb)


# speedup vs baseline: 5.4569x; 5.4569x over previous
import math

import jax
import jax.numpy as jnp
from jax import lax
from jax.experimental import pallas as pl

_BASE = 2
_BT = 2048


def _body(emb_ref, out_ref):
    l = pl.program_id(0)
    e = emb_ref[0]
    bt, d = out_ref.shape[1], out_ref.shape[2]
    tpar = lax.broadcasted_iota(jnp.int32, (bt, d), 0) % 2
    use_second = jnp.logical_and(tpar == 1, (l % _BASE) == 0)
    out_ref[0, :, :] = jnp.where(use_second, e[1:2, :], e[0:1, :])


def kernel(x, emb):
    B, T = x.shape
    del B
    max_level = int(math.ceil(math.log(T, _BASE)))
    d = emb.shape[1]
    emb3 = emb.reshape(emb.shape[0] // _BASE, _BASE, d)
    bt = min(_BT, T)
    grid = (max_level, T // bt)
    return pl.pallas_call(
        _body,
        grid=grid,
        in_specs=[pl.BlockSpec((1, _BASE, d), lambda l, j: (l, 0, 0))],
        out_specs=pl.BlockSpec((1, bt, d), lambda l, j: (l, j, 0)),
        out_shape=jax.ShapeDtypeStruct((max_level, T, d), emb.dtype),
    )(emb3)
